# Initial kernel scaffold; baseline (speedup 1.0000x reference)
import jax
import jax.numpy as jnp
from jax.experimental import pallas as pl


def kernel(x, edge_index, W_gc0, W_feat0, W_feat1, W_gc1, W_gc2):
    raise NotImplementedError("write your pallas kernel here")


# speedup vs baseline: 11.6640x; 11.6640x over previous
import functools

import jax
import jax.numpy as jnp
from jax import lax
from jax.experimental import pallas as pl
from jax.experimental.pallas import tpu as pltpu
from jax.experimental.pallas import tpu_sc as plsc

N = 10000
E = 160000
D = 128
H = 32
Z = 16

NC, NS = 2, 16
NW = NC * NS
EPW = E // NW
C = 1000
NCHUNK = EPW // C
RPT = 632


def _sc_mesh():
    return plsc.VectorSubcoreMesh(
        core_axis_name="c", subcore_axis_name="s", num_cores=NC, num_subcores=NS
    )


def _deg_body(dst_hbm, ones_hbm, zeros_hbm, out_hbm, didx_v, ones_v, acc_sh, sem):
    c = lax.axis_index("c")
    s = lax.axis_index("s")
    wid = c * NS + s

    @pl.when(s == 0)
    def _init():
        pltpu.sync_copy(zeros_hbm, acc_sh)

    pltpu.sync_copy(ones_hbm, ones_v)
    plsc.subcore_barrier()
    for k in range(NCHUNK):
        base = wid * EPW + k * C
        pltpu.sync_copy(dst_hbm.at[pl.ds(base, C)], didx_v)
        pltpu.sync_copy(ones_v, acc_sh.at[didx_v], add=True)
    plsc.subcore_barrier()

    @pl.when(s == 0)
    def _drain():
        pltpu.sync_copy(acc_sh, out_hbm.at[c])


def _degree_partials(dst, ones_c, zeros_n):
    fn = pl.kernel(
        _deg_body,
        out_type=jax.ShapeDtypeStruct((NC, N), jnp.float32),
        mesh=_sc_mesh(),
        scratch_types=[
            pltpu.VMEM((C,), jnp.int32),
            pltpu.VMEM((C,), jnp.float32),
            pltpu.VMEM_SHARED((N,), jnp.float32),
            pltpu.SemaphoreType.DMA,
        ],
        name="sc_degree",
        compiler_params=pltpu.CompilerParams(use_tc_tiling_on_sc=False),
    )
    return fn(dst, ones_c, zeros_n)


def _edge_body(F, h_hbm, src_hbm, dst_hbm, zeros_hbm, out_hbm,
               sidx_v, didx_v, rows_v, acc_sh, sem):
    c = lax.axis_index("c")
    s = lax.axis_index("s")
    wid = c * NS + s

    row0 = jnp.minimum(s * RPT, N - RPT)
    pltpu.sync_copy(zeros_hbm.at[pl.ds(row0, RPT)], acc_sh.at[pl.ds(row0, RPT)])
    plsc.subcore_barrier()
    for k in range(NCHUNK):
        base = wid * EPW + k * C
        pltpu.sync_copy(src_hbm.at[pl.ds(base, C)], sidx_v)
        pltpu.sync_copy(dst_hbm.at[pl.ds(base, C)], didx_v)
        pltpu.async_copy(h_hbm.at[sidx_v], rows_v, sem).wait()
        pltpu.sync_copy(rows_v, acc_sh.at[didx_v], add=True)
    plsc.subcore_barrier()
    pltpu.sync_copy(acc_sh.at[pl.ds(row0, RPT)],
                    out_hbm.at[c, pl.ds(row0, RPT)])


def _edge_pass(h, src, dst, zeros_nf, F):
    fn = pl.kernel(
        functools.partial(_edge_body, F),
        out_type=jax.ShapeDtypeStruct((NC, N, F), jnp.float32),
        mesh=_sc_mesh(),
        scratch_types=[
            pltpu.VMEM((C,), jnp.int32),
            pltpu.VMEM((C,), jnp.int32),
            pltpu.VMEM((C, F), jnp.float32),
            pltpu.VMEM_SHARED((N, F), jnp.float32),
            pltpu.SemaphoreType.DMA,
        ],
        name=f"sc_edge_pass_f{F}",
        compiler_params=pltpu.CompilerParams(use_tc_tiling_on_sc=False),
    )
    return fn(h, src, dst, zeros_nf)


R = 2048
GR = (N + R - 1) // R


def _dis_from(degp_ref):
    deg = jnp.maximum(degp_ref[0, :] + degp_ref[1, :], 1.0)
    return lax.rsqrt(deg)


def _stage0_body(x_ref, w0_ref, wf_ref, degp_ref, g0_ref, h1f_ref):
    dis = _dis_from(degp_ref)
    xw = jnp.dot(x_ref[...], w0_ref[...], preferred_element_type=jnp.float32)
    g0_ref[...] = xw * dis[:, None]
    h1f_ref[...] = jnp.maximum(
        jnp.dot(x_ref[...], wf_ref[...], preferred_element_type=jnp.float32), 0.0)


def _stage0(x, W_gc0, W_feat0, degp):
    return pl.pallas_call(
        _stage0_body,
        grid=(GR,),
        in_specs=[
            pl.BlockSpec((R, D), lambda i: (i, 0)),
            pl.BlockSpec((D, H), lambda i: (0, 0)),
            pl.BlockSpec((D, H), lambda i: (0, 0)),
            pl.BlockSpec((NC, R), lambda i: (0, i)),
        ],
        out_specs=[
            pl.BlockSpec((R, H), lambda i: (i, 0)),
            pl.BlockSpec((R, H), lambda i: (i, 0)),
        ],
        out_shape=[
            jax.ShapeDtypeStruct((N, H), jnp.float32),
            jax.ShapeDtypeStruct((N, H), jnp.float32),
        ],
    )(x, W_gc0, W_feat0, degp)


def _stage_mid_body(p_ref, degp_ref, h1f_ref, w_ref, g_ref):
    dis = _dis_from(degp_ref)
    agg = p_ref[0] + p_ref[1]
    h = jnp.maximum(agg * dis[:, None], 0.0) + h1f_ref[...]
    g_ref[...] = jnp.dot(h, w_ref[...], preferred_element_type=jnp.float32) * dis[:, None]


def _stage_mid(p, degp, h1f, W, Fout):
    return pl.pallas_call(
        _stage_mid_body,
        grid=(GR,),
        in_specs=[
            pl.BlockSpec((NC, R, H), lambda i: (0, i, 0)),
            pl.BlockSpec((NC, R), lambda i: (0, i)),
            pl.BlockSpec((R, H), lambda i: (i, 0)),
            pl.BlockSpec((H, Fout), lambda i: (0, 0)),
        ],
        out_specs=pl.BlockSpec((R, Fout), lambda i: (i, 0)),
        out_shape=jax.ShapeDtypeStruct((N, Fout), jnp.float32),
    )(p, degp, h1f, W)


BI = 1024
BJ = 512


def _rec_body(p_i, dg_i, p_j, dg_j, out_ref):
    def z_of(p_ref, dg_ref):
        dis = _dis_from(dg_ref)
        return (p_ref[0] + p_ref[1]) * dis[:, None]

    zi = z_of(p_i, dg_i)
    zj = z_of(p_j, dg_j)
    out_ref[...] = lax.dot_general(
        zi, zj, (((1,), (1,)), ((), ())), preferred_element_type=jnp.float32)


def _decoder(p3, degp):
    gi = (N + BI - 1) // BI
    gj = (N + BJ - 1) // BJ
    return pl.pallas_call(
        _rec_body,
        grid=(gi, gj),
        in_specs=[
            pl.BlockSpec((NC, BI, Z), lambda i, j: (0, i, 0)),
            pl.BlockSpec((NC, BI), lambda i, j: (0, i)),
            pl.BlockSpec((NC, BJ, Z), lambda i, j: (0, j, 0)),
            pl.BlockSpec((NC, BJ), lambda i, j: (0, j)),
        ],
        out_specs=pl.BlockSpec((BI, BJ), lambda i, j: (i, j)),
        out_shape=jax.ShapeDtypeStruct((N, N), jnp.float32),
    )(p3, degp, p3, degp)


def kernel(x, edge_index, W_gc0, W_feat0, W_feat1, W_gc1, W_gc2):
    src = edge_index[0]
    dst = edge_index[1]
    ones_c = jnp.ones((C,), jnp.float32)
    zeros_n = jnp.zeros((N,), jnp.float32)
    zeros_nh = jnp.zeros((N, H), jnp.float32)
    zeros_nz = jnp.zeros((N, Z), jnp.float32)

    degp = _degree_partials(dst, ones_c, zeros_n)
    g0, h1f = _stage0(x, W_gc0, W_feat0, degp)
    p1 = _edge_pass(g0, src, dst, zeros_nh, H)
    g1 = _stage_mid(p1, degp, h1f, W_gc1, H)
    p2 = _edge_pass(g1, src, dst, zeros_nh, H)
    g2 = _stage_mid(p2, degp, h1f, W_gc2, Z)
    p3 = _edge_pass(g2, src, dst, zeros_nz, Z)
    return _decoder(p3, degp)
